# baseline (speedup 1.0000x reference)
import functools

import jax
import jax.numpy as jnp
from jax import lax
from jax.experimental import pallas as pl
from jax.experimental.pallas import tpu as pltpu
from jax.experimental.pallas import tpu_sc as plsc

N = 10000
E = 320000
D = 128
K = 3

NC = 2
NS = 16
TILES = NC * NS
CHUNK = 125
HCHUNK = 50
EPT = E // TILES
CPT = EPT // CHUNK
HCPT = EPT // HCHUNK
HSTG = 40
NP = 10240
RPT = NP // NS
BLK = 1000
GRID = N // BLK

_MESH = plsc.VectorSubcoreMesh(core_axis_name="c", subcore_axis_name="s")


def _gelu(t):
    return 0.5 * t * (1.0 + lax.erf(t * 0.7071067811865476))


@functools.partial(
    pl.kernel,
    out_type=jax.ShapeDtypeStruct((NC * NP,), jnp.float32),
    mesh=_MESH,
    scratch_types=[
        pltpu.VMEM((CPT, CHUNK), jnp.int32),
        pltpu.VMEM((128,), jnp.float32),
        pltpu.VMEM_SHARED((NP,), jnp.float32),
    ],
)
def _sc_deg(dst_hbm, ones_hbm, zeros_hbm, out_hbm, dst_v, ones_v, acc_sh):
    cid = lax.axis_index("c")
    sid = lax.axis_index("s")
    tile = cid * NS + sid
    pltpu.sync_copy(dst_hbm.at[pl.ds(tile * CPT, CPT)], dst_v)
    pltpu.sync_copy(ones_hbm, ones_v)
    pltpu.sync_copy(zeros_hbm, acc_sh.at[pl.ds(sid * RPT, RPT)])
    plsc.subcore_barrier()

    def body(j, carry):
        pltpu.sync_copy(ones_v.at[pl.ds(0, CHUNK)], acc_sh.at[dst_v.at[j]],
                        add=True)
        return carry

    lax.fori_loop(0, CPT, body, 0)
    plsc.subcore_barrier()
    pltpu.sync_copy(acc_sh.at[pl.ds(sid * RPT, RPT)],
                    out_hbm.at[pl.ds(cid * NP + sid * RPT, RPT)])


@functools.partial(
    pl.kernel,
    out_type=jax.ShapeDtypeStruct((NC * NP, D), jnp.float32),
    mesh=_MESH,
    scratch_types=[
        pltpu.VMEM((HSTG, HCHUNK), jnp.int32),
        pltpu.VMEM((HSTG, HCHUNK), jnp.int32),
        pltpu.VMEM((4, HCHUNK, D), jnp.float32),
        pltpu.VMEM_SHARED((NP, D), jnp.float32),
        pltpu.SemaphoreType.DMA,
        pltpu.SemaphoreType.DMA,
        pltpu.SemaphoreType.DMA,
        pltpu.SemaphoreType.DMA,
        pltpu.SemaphoreType.DMA,
    ],
)
def _sc_hop(g_hbm, src_hbm, dst_hbm, zeros_hbm, out_hbm,
            src_v, dst_v, rows_v, acc_sh, gsem0, gsem1, gsem2, gsem3, ssem):
    cid = lax.axis_index("c")
    sid = lax.axis_index("s")
    tile = cid * NS + sid
    base = tile * HCPT
    gsems = (gsem0, gsem1, gsem2, gsem3)
    pltpu.sync_copy(zeros_hbm, acc_sh.at[pl.ds(sid * RPT, RPT)])
    plsc.subcore_barrier()

    def _gissue(j, b):
        pltpu.async_copy(g_hbm.at[src_v.at[j]], rows_v.at[b], gsems[b])

    def _gwait(j, b):
        pltpu.make_async_copy(g_hbm.at[src_v.at[j]], rows_v.at[b],
                              gsems[b]).wait()

    def _sissue(j, b):
        pltpu.async_copy(rows_v.at[b], acc_sh.at[dst_v.at[j]], ssem, add=True)

    def _swait(j, b):
        pltpu.make_async_copy(rows_v.at[b], acc_sh.at[dst_v.at[j]],
                              ssem).wait()

    for s in range(HCPT // HSTG):
        pltpu.sync_copy(src_hbm.at[pl.ds(base + s * HSTG, HSTG)], src_v)
        pltpu.sync_copy(dst_hbm.at[pl.ds(base + s * HSTG, HSTG)], dst_v)
        _gissue(0, 0)
        _gissue(1, 1)
        _gissue(2, 2)

        def body(i, carry):
            j0 = i * 4
            for t in range(4):
                j = j0 + t
                _gwait(j, t)

                @pl.when(j > 0)
                def _():
                    _swait(j - 1, (t - 1) % 4)

                _sissue(j, t)

                @pl.when(j + 3 < HSTG)
                def _():
                    _gissue(j + 3, (t + 3) % 4)

            return carry

        lax.fori_loop(0, HSTG // 4, body, 0)
        _swait(HSTG - 1, 3)
    plsc.subcore_barrier()
    pltpu.sync_copy(acc_sh.at[pl.ds(sid * RPT, RPT)],
                    out_hbm.at[pl.ds(cid * NP + sid * RPT, RPT)])


def _tc_main(x, gamma, beta, lin_W, lin_b, W0, degp):
    def body(x_ref, ga_ref, be_ref, lw_ref, lb_ref, w0_ref, dp_ref,
             xb_ref, out0_ref, g_ref, dinv_ref, st_ref):
        ph = pl.program_id(0)
        i = pl.program_id(1)

        @pl.when(ph == 0)
        def _():
            xv = x_ref[...]
            blk = jnp.stack([jnp.sum(xv, axis=0), jnp.sum(xv * xv, axis=0)])

            @pl.when(i == 0)
            def _():
                st_ref[...] = blk

            @pl.when(i > 0)
            def _():
                st_ref[...] = st_ref[...] + blk

        @pl.when(ph == 1)
        def _():
            mean = st_ref[0:1, :] * (1.0 / N)
            ex2 = st_ref[1:2, :] * (1.0 / N)
            var = ex2 - mean * mean
            scale = lax.rsqrt(var + 1e-5) * ga_ref[...]
            xb = (x_ref[...] - mean) * scale + be_ref[...]
            h = _gelu(jnp.dot(xb, lw_ref[...],
                              preferred_element_type=jnp.float32) + lb_ref[...])
            deg = dp_ref[:, 0:1] + dp_ref[:, 1:2]
            dinv = jnp.where(deg > 0, lax.rsqrt(deg), 0.0)
            xb_ref[...] = xb
            out0_ref[...] = jnp.dot(h, w0_ref[...],
                                    preferred_element_type=jnp.float32)
            g_ref[...] = h * dinv
            dinv_ref[...] = dinv

    full = lambda s: pl.BlockSpec(s, lambda p, i: (0,) * len(s))
    return pl.pallas_call(
        body,
        grid=(2, GRID),
        in_specs=[
            pl.BlockSpec((BLK, D), lambda p, i: (i, 0)),
            full((1, D)), full((1, D)),
            full((D, D)), full((1, D)), full((D, D)),
            pl.BlockSpec((BLK, 2), lambda p, i: (i, 0)),
        ],
        out_specs=[
            pl.BlockSpec((BLK, D), lambda p, i: (i, 0)),
            pl.BlockSpec((BLK, D), lambda p, i: (i, 0)),
            pl.BlockSpec((BLK, D), lambda p, i: (i, 0)),
            pl.BlockSpec((BLK, 1), lambda p, i: (i, 0)),
        ],
        out_shape=[
            jax.ShapeDtypeStruct((N, D), jnp.float32),
            jax.ShapeDtypeStruct((N, D), jnp.float32),
            jax.ShapeDtypeStruct((N, D), jnp.float32),
            jax.ShapeDtypeStruct((N, 1), jnp.float32),
        ],
        scratch_shapes=[pltpu.VMEM((2, D), jnp.float32)],
    )(x, gamma, beta, lin_W, lin_b, W0, degp)


def _tc_hop(p, dinv, outp, Wk):
    def body(p_ref, di_ref, op_ref, wk_ref, out_ref, g_ref):
        dinv_v = di_ref[...]
        hk = (p_ref[0] + p_ref[1]) * dinv_v
        out_ref[...] = op_ref[...] + jnp.dot(hk, wk_ref[...],
                                             preferred_element_type=jnp.float32)
        g_ref[...] = hk * dinv_v

    return pl.pallas_call(
        body,
        grid=(GRID,),
        in_specs=[
            pl.BlockSpec((2, BLK, D), lambda i: (0, i, 0)),
            pl.BlockSpec((BLK, 1), lambda i: (i, 0)),
            pl.BlockSpec((BLK, D), lambda i: (i, 0)),
            pl.BlockSpec((D, D), lambda i: (0, 0)),
        ],
        out_specs=[
            pl.BlockSpec((BLK, D), lambda i: (i, 0)),
            pl.BlockSpec((BLK, D), lambda i: (i, 0)),
        ],
        out_shape=[
            jax.ShapeDtypeStruct((N, D), jnp.float32),
            jax.ShapeDtypeStruct((N, D), jnp.float32),
        ],
    )(p, dinv, outp, Wk)


def _tc_final(p, dinv, outp, W3, tag_b, xb, W1, b1, W2, b2, Wc, bc):
    def body(p_ref, di_ref, op_ref, w3_ref, tb_ref, xb_ref,
             w1_ref, b1_ref, w2_ref, b2_ref, wc_ref, bc_ref, a_ref, cl_ref):
        hk = (p_ref[0] + p_ref[1]) * di_ref[...]
        out = op_ref[...] + jnp.dot(hk, w3_ref[...],
                                    preferred_element_type=jnp.float32)
        h2 = _gelu(out + tb_ref[...])
        w1 = w1_ref[...]
        a1 = (jnp.dot(_gelu(xb_ref[...]), w1[:D],
                      preferred_element_type=jnp.float32)
              + jnp.dot(_gelu(h2), w1[D:],
                        preferred_element_type=jnp.float32)
              + b1_ref[...])
        a2 = jnp.dot(_gelu(a1), w2_ref[...],
                     preferred_element_type=jnp.float32) + b2_ref[...]
        cl = jnp.dot(_gelu(a2), wc_ref[...],
                     preferred_element_type=jnp.float32) + bc_ref[...]
        a_ref[...] = a2
        cl_ref[...] = cl

    full = lambda s: pl.BlockSpec(s, lambda i: (0,) * len(s))
    return pl.pallas_call(
        body,
        grid=(GRID,),
        in_specs=[
            pl.BlockSpec((2, BLK, D), lambda i: (0, i, 0)),
            pl.BlockSpec((BLK, 1), lambda i: (i, 0)),
            pl.BlockSpec((BLK, D), lambda i: (i, 0)),
            full((D, D)), full((1, D)),
            pl.BlockSpec((BLK, D), lambda i: (i, 0)),
            full((2 * D, D)), full((1, D)),
            full((D, 64)), full((1, 64)),
            full((64, 16)), full((1, 16)),
        ],
        out_specs=[
            pl.BlockSpec((BLK, 64), lambda i: (i, 0)),
            pl.BlockSpec((BLK, 16), lambda i: (i, 0)),
        ],
        out_shape=[
            jax.ShapeDtypeStruct((N, 64), jnp.float32),
            jax.ShapeDtypeStruct((N, 16), jnp.float32),
        ],
    )(p, dinv, outp, W3, tag_b, xb, W1, b1, W2, b2, Wc, bc)


def kernel(x, bn_gamma, bn_beta, lin_W, lin_b, tag_Ws, tag_b,
           W1, b1, W2, b2, Wc, bc, edge_index):
    src2d = edge_index[0].reshape(E // HCHUNK, HCHUNK)
    dst2d = edge_index[1].reshape(E // HCHUNK, HCHUNK)
    dst2d_deg = edge_index[1].reshape(E // CHUNK, CHUNK)
    ones1 = jnp.ones((128,), jnp.float32)
    zeros1 = jnp.zeros((RPT,), jnp.float32)
    zerosD = jnp.zeros((RPT, D), jnp.float32)

    degp = _sc_deg(dst2d_deg, ones1, zeros1).reshape(NC, NP)[:, :N].T
    xb, outk, g, dinv = _tc_main(x, bn_gamma[None, :], bn_beta[None, :],
                                 lin_W, lin_b[None, :], tag_Ws[0], degp)
    for k in range(1, K + 1):
        p = _sc_hop(g, src2d, dst2d, zerosD).reshape(NC, NP, D)
        if k < K:
            outk, g = _tc_hop(p, dinv, outk, tag_Ws[k])
        else:
            a, cl = _tc_final(p, dinv, outk, tag_Ws[k], tag_b[None, :], xb,
                              W1, b1[None, :], W2, b2[None, :], Wc, bc[None, :])
    return (a, cl)


# speedup vs baseline: 1.0182x; 1.0182x over previous
import functools

import jax
import jax.numpy as jnp
from jax import lax
from jax.experimental import pallas as pl
from jax.experimental.pallas import tpu as pltpu
from jax.experimental.pallas import tpu_sc as plsc

N = 10000
E = 320000
D = 128
K = 3

NC = 2
NS = 16
TILES = NC * NS
CHUNK = 125
HCHUNK = 50
EPT = E // TILES
CPT = EPT // CHUNK
HCPT = EPT // HCHUNK
HSTG = 40
NP = 10240
RPT = NP // NS
BLK = 1000
GRID = N // BLK

_MESH = plsc.VectorSubcoreMesh(core_axis_name="c", subcore_axis_name="s")


def _gelu(t):
    return 0.5 * t * (1.0 + lax.erf(t * 0.7071067811865476))


@functools.partial(
    pl.kernel,
    out_type=jax.ShapeDtypeStruct((NC * NP,), jnp.float32),
    mesh=_MESH,
    scratch_types=[
        pltpu.VMEM((CPT, CHUNK), jnp.int32),
        pltpu.VMEM((128,), jnp.float32),
        pltpu.VMEM_SHARED((NP,), jnp.float32),
    ],
)
def _sc_deg(dst_hbm, ones_hbm, zeros_hbm, out_hbm, dst_v, ones_v, acc_sh):
    cid = lax.axis_index("c")
    sid = lax.axis_index("s")
    tile = cid * NS + sid
    pltpu.sync_copy(dst_hbm.at[pl.ds(tile * CPT, CPT)], dst_v)
    pltpu.sync_copy(ones_hbm, ones_v)
    pltpu.sync_copy(zeros_hbm, acc_sh.at[pl.ds(sid * RPT, RPT)])
    plsc.subcore_barrier()

    def body(j, carry):
        pltpu.sync_copy(ones_v.at[pl.ds(0, CHUNK)], acc_sh.at[dst_v.at[j]],
                        add=True)
        return carry

    lax.fori_loop(0, CPT, body, 0)
    plsc.subcore_barrier()
    pltpu.sync_copy(acc_sh.at[pl.ds(sid * RPT, RPT)],
                    out_hbm.at[pl.ds(cid * NP + sid * RPT, RPT)])


@functools.partial(
    pl.kernel,
    out_type=jax.ShapeDtypeStruct((NC * NP, D), jnp.float32),
    mesh=_MESH,
    scratch_types=[
        pltpu.VMEM((HSTG, HCHUNK), jnp.int32),
        pltpu.VMEM((HSTG, HCHUNK), jnp.int32),
        pltpu.VMEM((4, HCHUNK, D), jnp.float32),
        pltpu.VMEM_SHARED((NP, D), jnp.float32),
        pltpu.SemaphoreType.DMA,
        pltpu.SemaphoreType.DMA,
        pltpu.SemaphoreType.DMA,
        pltpu.SemaphoreType.DMA,
        pltpu.SemaphoreType.DMA,
    ],
)
def _sc_hop(g_hbm, src_hbm, dst_hbm, zeros_hbm, out_hbm,
            src_v, dst_v, rows_v, acc_sh, gsem0, gsem1, gsem2, gsem3, ssem):
    cid = lax.axis_index("c")
    sid = lax.axis_index("s")
    tile = cid * NS + sid
    base = tile * HCPT
    gsems = (gsem0, gsem1, gsem2, gsem3)
    pltpu.sync_copy(zeros_hbm, acc_sh.at[pl.ds(sid * RPT, RPT)])
    plsc.subcore_barrier()

    def _gissue(j, b):
        pltpu.async_copy(g_hbm.at[src_v.at[j]], rows_v.at[b], gsems[b])

    def _gwait(j, b):
        pltpu.make_async_copy(g_hbm.at[src_v.at[j]], rows_v.at[b],
                              gsems[b]).wait()

    def _sissue(j, b):
        pltpu.async_copy(rows_v.at[b], acc_sh.at[dst_v.at[j]], ssem, add=True)

    def _swait(j, b):
        pltpu.make_async_copy(rows_v.at[b], acc_sh.at[dst_v.at[j]],
                              ssem).wait()

    for s in range(HCPT // HSTG):
        pltpu.sync_copy(src_hbm.at[pl.ds(base + s * HSTG, HSTG)], src_v)
        pltpu.sync_copy(dst_hbm.at[pl.ds(base + s * HSTG, HSTG)], dst_v)
        _gissue(0, 0)
        _gissue(1, 1)
        _gissue(2, 2)

        def body(i, carry):
            j0 = i * 4
            for t in range(4):
                j = j0 + t
                _gwait(j, t)

                @pl.when(j > 0)
                def _():
                    _swait(j - 1, (t - 1) % 4)

                _sissue(j, t)

                @pl.when(j + 3 < HSTG)
                def _():
                    _gissue(j + 3, (t + 3) % 4)

            return carry

        lax.fori_loop(0, HSTG // 4, body, 0)
        _swait(HSTG - 1, 3)
    plsc.subcore_barrier()
    pltpu.sync_copy(acc_sh.at[pl.ds(sid * RPT, RPT)],
                    out_hbm.at[pl.ds(cid * NP + sid * RPT, RPT)])


def _tc_stats(x):
    def body(x_ref, o_ref):
        xv = x_ref[...]
        o_ref[...] = jnp.stack([jnp.sum(xv, axis=0), jnp.sum(xv * xv, axis=0)])

    return pl.pallas_call(
        body, out_shape=jax.ShapeDtypeStruct((2, D), jnp.float32))(x)


def _tc_main(x, stats, gamma, beta, lin_W, lin_b, W0, degp):
    def body(x_ref, st_ref, ga_ref, be_ref, lw_ref, lb_ref, w0_ref, dp_ref,
             xb_ref, out0_ref, g_ref, dinv_ref):
        mean = st_ref[0:1, :] * (1.0 / N)
        ex2 = st_ref[1:2, :] * (1.0 / N)
        var = ex2 - mean * mean
        scale = lax.rsqrt(var + 1e-5) * ga_ref[...]
        xb = (x_ref[...] - mean) * scale + be_ref[...]
        h = _gelu(jnp.dot(xb, lw_ref[...], preferred_element_type=jnp.float32)
                  + lb_ref[...])
        deg = dp_ref[:, 0:1] + dp_ref[:, 1:2]
        dinv = jnp.where(deg > 0, lax.rsqrt(deg), 0.0)
        xb_ref[...] = xb
        out0_ref[...] = jnp.dot(h, w0_ref[...], preferred_element_type=jnp.float32)
        g_ref[...] = h * dinv
        dinv_ref[...] = dinv

    full = lambda s: pl.BlockSpec(s, lambda i: (0,) * len(s))
    return pl.pallas_call(
        body,
        grid=(GRID,),
        in_specs=[
            pl.BlockSpec((BLK, D), lambda i: (i, 0)),
            full((2, D)), full((1, D)), full((1, D)),
            full((D, D)), full((1, D)), full((D, D)),
            pl.BlockSpec((BLK, 2), lambda i: (i, 0)),
        ],
        out_specs=[
            pl.BlockSpec((BLK, D), lambda i: (i, 0)),
            pl.BlockSpec((BLK, D), lambda i: (i, 0)),
            pl.BlockSpec((BLK, D), lambda i: (i, 0)),
            pl.BlockSpec((BLK, 1), lambda i: (i, 0)),
        ],
        out_shape=[
            jax.ShapeDtypeStruct((N, D), jnp.float32),
            jax.ShapeDtypeStruct((N, D), jnp.float32),
            jax.ShapeDtypeStruct((N, D), jnp.float32),
            jax.ShapeDtypeStruct((N, 1), jnp.float32),
        ],
    )(x, stats, gamma, beta, lin_W, lin_b, W0, degp)


def _tc_hop(p, dinv, outp, Wk):
    def body(p_ref, di_ref, op_ref, wk_ref, out_ref, g_ref):
        dinv_v = di_ref[...]
        hk = (p_ref[0] + p_ref[1]) * dinv_v
        out_ref[...] = op_ref[...] + jnp.dot(hk, wk_ref[...],
                                             preferred_element_type=jnp.float32)
        g_ref[...] = hk * dinv_v

    return pl.pallas_call(
        body,
        grid=(GRID,),
        in_specs=[
            pl.BlockSpec((2, BLK, D), lambda i: (0, i, 0)),
            pl.BlockSpec((BLK, 1), lambda i: (i, 0)),
            pl.BlockSpec((BLK, D), lambda i: (i, 0)),
            pl.BlockSpec((D, D), lambda i: (0, 0)),
        ],
        out_specs=[
            pl.BlockSpec((BLK, D), lambda i: (i, 0)),
            pl.BlockSpec((BLK, D), lambda i: (i, 0)),
        ],
        out_shape=[
            jax.ShapeDtypeStruct((N, D), jnp.float32),
            jax.ShapeDtypeStruct((N, D), jnp.float32),
        ],
    )(p, dinv, outp, Wk)


def _tc_final(p, dinv, outp, W3, tag_b, xb, W1, b1, W2, b2, Wc, bc):
    def body(p_ref, di_ref, op_ref, w3_ref, tb_ref, xb_ref,
             w1_ref, b1_ref, w2_ref, b2_ref, wc_ref, bc_ref, a_ref, cl_ref):
        hk = (p_ref[0] + p_ref[1]) * di_ref[...]
        out = op_ref[...] + jnp.dot(hk, w3_ref[...],
                                    preferred_element_type=jnp.float32)
        h2 = _gelu(out + tb_ref[...])
        w1 = w1_ref[...]
        a1 = (jnp.dot(_gelu(xb_ref[...]), w1[:D],
                      preferred_element_type=jnp.float32)
              + jnp.dot(_gelu(h2), w1[D:],
                        preferred_element_type=jnp.float32)
              + b1_ref[...])
        a2 = jnp.dot(_gelu(a1), w2_ref[...],
                     preferred_element_type=jnp.float32) + b2_ref[...]
        cl = jnp.dot(_gelu(a2), wc_ref[...],
                     preferred_element_type=jnp.float32) + bc_ref[...]
        a_ref[...] = a2
        cl_ref[...] = cl

    full = lambda s: pl.BlockSpec(s, lambda i: (0,) * len(s))
    return pl.pallas_call(
        body,
        grid=(GRID,),
        in_specs=[
            pl.BlockSpec((2, BLK, D), lambda i: (0, i, 0)),
            pl.BlockSpec((BLK, 1), lambda i: (i, 0)),
            pl.BlockSpec((BLK, D), lambda i: (i, 0)),
            full((D, D)), full((1, D)),
            pl.BlockSpec((BLK, D), lambda i: (i, 0)),
            full((2 * D, D)), full((1, D)),
            full((D, 64)), full((1, 64)),
            full((64, 16)), full((1, 16)),
        ],
        out_specs=[
            pl.BlockSpec((BLK, 64), lambda i: (i, 0)),
            pl.BlockSpec((BLK, 16), lambda i: (i, 0)),
        ],
        out_shape=[
            jax.ShapeDtypeStruct((N, 64), jnp.float32),
            jax.ShapeDtypeStruct((N, 16), jnp.float32),
        ],
    )(p, dinv, outp, W3, tag_b, xb, W1, b1, W2, b2, Wc, bc)


def kernel(x, bn_gamma, bn_beta, lin_W, lin_b, tag_Ws, tag_b,
           W1, b1, W2, b2, Wc, bc, edge_index):
    src2d = edge_index[0].reshape(E // HCHUNK, HCHUNK)
    dst2d = edge_index[1].reshape(E // HCHUNK, HCHUNK)
    dst2d_deg = edge_index[1].reshape(E // CHUNK, CHUNK)
    ones1 = jnp.ones((128,), jnp.float32)
    zeros1 = jnp.zeros((RPT,), jnp.float32)
    zerosD = jnp.zeros((RPT, D), jnp.float32)

    degp = _sc_deg(dst2d_deg, ones1, zeros1).reshape(NC, NP)[:, :N].T
    stats = _tc_stats(x)
    xb, outk, g, dinv = _tc_main(x, stats, bn_gamma[None, :], bn_beta[None, :],
                                 lin_W, lin_b[None, :], tag_Ws[0], degp)
    for k in range(1, K + 1):
        p = _sc_hop(g, src2d, dst2d, zerosD).reshape(NC, NP, D)
        if k < K:
            outk, g = _tc_hop(p, dinv, outk, tag_Ws[k])
        else:
            a, cl = _tc_final(p, dinv, outk, tag_Ws[k], tag_b[None, :], xb,
                              W1, b1[None, :], W2, b2[None, :], Wc, bc[None, :])
    return (a, cl)
